# Initial kernel scaffold; baseline (speedup 1.0000x reference)
import jax
import jax.numpy as jnp
from jax.experimental import pallas as pl


def kernel(heatmap, offset, log_flux, gt_centroids, gt_log_flux):
    raise NotImplementedError("write your pallas kernel here")


# speedup vs baseline: 10.7179x; 10.7179x over previous
import functools

import jax
import jax.numpy as jnp
from jax.experimental import pallas as pl
from jax.experimental.pallas import tpu as pltpu

_LAMBDA_HM = 1.0
_LAMBDA_OFF = 1.0
_LAMBDA_FLUX = 0.1
_SIGMA = 2.0

_H = 256
_W = 256
_K = 64
_PAD = 8


def _tc_body(hm_ref, off_ref, flux_ref, gtc_ref, gtct_ref, gtf_ref,
             out_ref, canvas_ref):
    H, W, K = _H, _W, _K
    radius = float(int(3 * _SIGMA + 1))
    inv2s2 = 2.0 * _SIGMA ** 2

    cx = gtc_ref[0, :, 0:1] * float(W - 1)
    cy = gtc_ref[0, :, 1:2] * float(H - 1)
    cxi_f = jnp.clip(jnp.round(cx), 0.0, float(W - 1))
    cyi_f = jnp.clip(jnp.round(cy), 0.0, float(H - 1))
    cxi = cxi_f.astype(jnp.int32)
    cyi = cyi_f.astype(jnp.int32)
    dxk = cx - cxi_f
    dyk = cy - cyi_f
    enc_col = cyi * W + cxi

    cx_r = gtct_ref[0, 0:1, :] * float(W - 1)
    cy_r = gtct_ref[0, 1:2, :] * float(H - 1)
    cxi_r = jnp.clip(jnp.round(cx_r), 0.0, float(W - 1)).astype(jnp.int32)
    cyi_r = jnp.clip(jnp.round(cy_r), 0.0, float(H - 1)).astype(jnp.int32)
    enc_row = cyi_r * W + cxi_r

    canvas_ref[...] = jnp.zeros((H + 3 * _PAD, W), jnp.float32)
    r_io = jax.lax.broadcasted_iota(jnp.int32, (24, W), 0).astype(jnp.float32)
    x_io = jax.lax.broadcasted_iota(jnp.int32, (24, W), 1).astype(jnp.float32)
    for k in range(K):
        cx_k = cx[k, 0]
        cy_k = cy[k, 0]
        cxi_kf = cxi_f[k, 0]
        cyi_kf = cyi_f[k, 0]
        y0 = cyi[k, 0] + (_PAD - 7)
        s0 = pl.multiple_of((y0 // 8) * 8, 8)
        yf = (s0 - _PAD).astype(jnp.float32) + r_io
        g = jnp.exp(-(((x_io - cx_k) ** 2) + ((yf - cy_k) ** 2)) / inv2s2)
        win = (jnp.abs(x_io - cxi_kf) <= radius) & (jnp.abs(yf - cyi_kf) <= radius)
        g = jnp.where(win, g, 0.0)
        cur = canvas_ref[pl.ds(s0, 24), :]
        canvas_ref[pl.ds(s0, 24), :] = jnp.maximum(cur, g)

    p = jnp.clip(hm_ref[0], 1e-6, 1.0 - 1e-6)
    t = canvas_ref[_PAD:_PAD + H, :]
    pos = t == 1.0
    one_m_p = 1.0 - p
    pos_l = -(one_m_p * one_m_p) * jnp.log(p)
    omt = 1.0 - t
    omt2 = omt * omt
    neg_l = -(omt2 * omt2) * (p * p) * jnp.log(1.0 - p)
    s_f = jnp.sum(jnp.where(pos, pos_l, neg_l))
    pc = jnp.sum(pos.astype(jnp.float32))

    io_i = jax.lax.broadcasted_iota(jnp.int32, (K, K), 0)
    io_j = jax.lax.broadcasted_iota(jnp.int32, (K, K), 1)
    dup_later = jnp.any((enc_col == enc_row) & (io_j > io_i), axis=1,
                        keepdims=True)
    winner = jnp.where(dup_later, 0.0, 1.0)

    col_io = jax.lax.broadcasted_iota(jnp.int32, (K, W), 1)
    row_io = col_io
    oh_y = (row_io == cyi).astype(jnp.float32)
    oh_x = (col_io == cxi).astype(jnp.float32)

    def _gather(img):
        rows = jax.lax.dot_general(
            oh_y, img, (((1,), (0,)), ((), ())),
            preferred_element_type=jnp.float32)
        return jnp.sum(rows * oh_x, axis=1, keepdims=True)

    v0 = _gather(off_ref[0, 0])
    v1 = _gather(off_ref[0, 1])
    vf = _gather(flux_ref[0])
    s_off = jnp.sum(winner * (jnp.abs(v0 - dxk) + jnp.abs(v1 - dyk)))
    s_flux = jnp.sum(winner * jnp.abs(vf - gtf_ref[0]))
    n_pos = jnp.sum(winner)

    lane = jax.lax.broadcasted_iota(jnp.int32, (1, 1, 128), 2)
    vals = (jnp.where(lane == 0, s_f, 0.0)
            + jnp.where(lane == 1, pc, 0.0)
            + jnp.where(lane == 2, s_off, 0.0)
            + jnp.where(lane == 3, s_flux, 0.0)
            + jnp.where(lane == 4, n_pos, 0.0))
    out_ref[...] = vals


@functools.partial(jax.jit, static_argnames=("interpret",))
def _run_tc(hm, offset, flux, gtc, gtct, gtf, interpret=False):
    B = hm.shape[0]
    return pl.pallas_call(
        _tc_body,
        grid=(B,),
        in_specs=[
            pl.BlockSpec((1, _H, _W), lambda i: (i, 0, 0)),
            pl.BlockSpec((1, 2, _H, _W), lambda i: (i, 0, 0, 0)),
            pl.BlockSpec((1, _H, _W), lambda i: (i, 0, 0)),
            pl.BlockSpec((1, _K, 2), lambda i: (i, 0, 0)),
            pl.BlockSpec((1, 2, _K), lambda i: (i, 0, 0)),
            pl.BlockSpec((1, _K, 1), lambda i: (i, 0, 0)),
        ],
        out_specs=pl.BlockSpec((1, 1, 128), lambda i: (i, 0, 0)),
        out_shape=jax.ShapeDtypeStruct((B, 1, 128), jnp.float32),
        scratch_shapes=[pltpu.VMEM((_H + 3 * _PAD, _W), jnp.float32)],
        interpret=interpret,
    )(hm, offset, flux, gtc, gtct, gtf)


def kernel(heatmap, offset, log_flux, gt_centroids, gt_log_flux,
           interpret=False):
    K = gt_centroids.shape[1]
    hm = heatmap[:, 0]
    gtct = jnp.swapaxes(gt_centroids, 1, 2)
    gtf = gt_log_flux[:, :, None]
    out = _run_tc(hm, offset, log_flux, gt_centroids, gtct, gtf,
                  interpret=interpret)
    s_f = jnp.sum(out[:, 0, 0])
    pc = jnp.sum(out[:, 0, 1])
    s_off = jnp.sum(out[:, 0, 2])
    s_flux = jnp.sum(out[:, 0, 3])
    n_pos = jnp.sum(out[:, 0, 4])
    loss_hm = s_f / jnp.maximum(pc, 1.0)
    n_pos_c = jnp.maximum(n_pos, 1.0)
    l_hm = _LAMBDA_HM * loss_hm
    l_off = _LAMBDA_OFF * (s_off / n_pos_c)
    l_fl = _LAMBDA_FLUX * (s_flux / n_pos_c)
    total = l_hm + l_off + l_fl
    return (l_hm, l_off, l_fl, total, jnp.asarray(float(K), jnp.float32))
